# baseline (speedup 1.0000x reference)
import functools

import jax
import jax.numpy as jnp
from jax import lax
from jax.experimental import pallas as pl
from jax.experimental.pallas import tpu as pltpu
from jax.experimental.pallas import tpu_sc as plsc

_N_ROWS = 1000000
_DIM = 16
_BATCH = 16384
_MOM = 0.5
_NC, _NS, _L = 2, 16, 16
_NW = _NC * _NS
_RPW = _N_ROWS // _NW
_SPAN = 31248
_WCAP = 31264
_NGB = _BATCH // _L
_HCAP = _BATCH + 2 * _L
_CR = 336
_CE = _CR * _DIM
_NCH = _SPAN // _CR
_SENT = 0x7FFFFFFF


def _bsplat(s):
    return lax.broadcast_in_dim(s, (_L,), ())


def _row_normalize(v):
    s = jnp.sum(v * v)
    sb = _bsplat(s)
    i = plsc.bitcast(sb, jnp.int32)
    i = 0x5F3759DF - lax.shift_right_arithmetic(i, 1)
    y = plsc.bitcast(i, jnp.float32)
    for _ in range(3):
        y = y * (1.5 - 0.5 * sb * y * y)
    norm = sb * y
    return v / (norm + 1e-10)


def _mb_body(feats_hbm, idx_hbm, bank_hbm, out_hbm,
             idxv, wref, hitw, hitrow, tmp,
             cb0, cb1, cb2, fsp,
             sem_f, sem_i0, sem_i1, sem_i2, sem_o0, sem_o1, sem_o2):
    cid = lax.axis_index("c")
    sid = lax.axis_index("s")
    wid = cid * _NS + sid
    lo_r = (wid * _RPW) // 8 * 8
    hi_r = ((wid + 1) * _RPW) // 8 * 8
    lo_e = pl.multiple_of(lo_r * _DIM, 128)

    iota = lax.iota(jnp.int32, _L)
    neg1 = jnp.full((_L,), -1, jnp.int32)

    @pl.when(sid == 0)
    def _stage():
        pltpu.async_copy(feats_hbm, fsp, sem_f)

    pltpu.sync_copy(idx_hbm, idxv)

    def read_scalar(ref, i):
        g = ref[pl.ds((i >> 4) << 4, _L)]
        return jnp.sum(jnp.where(iota == (i & 15), g, 0))

    def ib(g, c):
        wref[pl.ds(g * _L, _L)] = neg1
        return c
    lax.fori_loop(0, _WCAP // _L, ib, 0)

    def claim(g, c):
        t = idxv[pl.ds(g * _L, _L)]
        m = (t >= lo_r) & (t < hi_r)
        local = jnp.where(m, t - lo_r, 0)
        bvec = g * _L + iota
        cnt = jnp.sum(jnp.where(m, 1, 0))

        def fast(c):
            plsc.store_scatter(wref, [local], bvec, mask=m)
            return c

        def slow(c):
            for lane in range(_L):
                plsc.store_scatter(wref, [local], bvec,
                                   mask=m & (iota == lane))
            return c

        return lax.cond(
            cnt > 1, slow,
            lambda c: lax.cond(cnt > 0, fast, lambda c: c, c), c)
    lax.fori_loop(0, _NGB, claim, 0)

    def collect(g, cur):
        wv = wref[pl.ds(g * _L, _L)]
        m = wv >= 0
        cnt = jnp.sum(jnp.where(m, 1, 0))
        rowv = g * _L + iota

        def do(cur):
            plsc.store_compressed(hitw.at[pl.ds(cur, _L)], wv, mask=m)
            plsc.store_compressed(hitrow.at[pl.ds(cur, _L)], rowv, mask=m)
            return cur + cnt

        return lax.cond(cnt > 0, do, lambda c: c, cur)
    n = lax.fori_loop(0, _WCAP // _L, collect, 0)

    sbase = (n >> 4) << 4
    sg = hitrow[pl.ds(sbase, _L)]
    hitrow[pl.ds(sbase, _L)] = jnp.where(sbase + iota >= n, _SENT, sg)

    @pl.when(sid == 0)
    def _stage_wait():
        pltpu.make_async_copy(feats_hbm, fsp, sem_f).wait()
    plsc.subcore_barrier()

    bufs = (cb0, cb1, cb2)
    isems = (sem_i0, sem_i1, sem_i2)
    osems = (sem_o0, sem_o1, sem_o2)

    def chunk_src(j):
        return bank_hbm.at[pl.ds(pl.multiple_of(lo_e + j * _CE, 128), _CE)]

    def chunk_dst(j):
        return out_hbm.at[pl.ds(pl.multiple_of(lo_e + j * _CE, 128), _CE)]

    def process(buf, base, hp, row):
        def wcond(c):
            hp_, row_ = c
            return row_ < base + _CR

        def wbody(c):
            hp_, row_ = c
            b = read_scalar(hitw, hp_)
            pltpu.sync_copy(fsp.at[pl.ds(b * _DIM, _DIM)], tmp)
            f = _row_normalize(tmp[pl.ds(0, _DIM)])
            off = (row_ - base) * _DIM
            old = buf[pl.ds(off, _DIM)]
            blended = (1.0 - _MOM) * old + _MOM * f
            buf[pl.ds(off, _DIM)] = _row_normalize(blended)
            hp2 = hp_ + 1
            return hp2, read_scalar(hitrow, hp2)
        return lax.while_loop(wcond, wbody, (hp, row))

    pltpu.async_copy(chunk_src(0), cb0, sem_i0)
    row0 = read_scalar(hitrow, 0)

    def triple(i, carry):
        hp, row = carry
        for k in range(3):
            j = i * 3 + k
            ks = (k + 1) % 3

            @pl.when(j >= 2)
            def _drain():
                pltpu.make_async_copy(bufs[ks], chunk_dst(j - 2),
                                      osems[ks]).wait()

            @pl.when(j + 1 < _NCH)
            def _prefetch():
                pltpu.async_copy(chunk_src(j + 1), bufs[ks], isems[ks])

            pltpu.make_async_copy(chunk_src(j), bufs[k], isems[k]).wait()
            hp, row = process(bufs[k], j * _CR, hp, row)
            pltpu.async_copy(bufs[k], chunk_dst(j), osems[k])
        return hp, row

    hp, row = lax.fori_loop(0, _NCH // 3, triple, (jnp.int32(0), row0))

    for j in (_NCH - 2, _NCH - 1):
        pltpu.make_async_copy(bufs[j % 3], chunk_dst(j), osems[j % 3]).wait()

    @pl.when(hi_r - lo_r > _SPAN)
    def _tail():
        te = pl.multiple_of(lo_e + _SPAN * _DIM, 128)
        pltpu.sync_copy(bank_hbm.at[pl.ds(te, 8 * _DIM)],
                        cb0.at[pl.ds(0, 8 * _DIM)])
        process(cb0, _SPAN, hp, row)
        pltpu.sync_copy(cb0.at[pl.ds(0, 8 * _DIM)],
                        out_hbm.at[pl.ds(te, 8 * _DIM)])


_mb_update = functools.partial(
    pl.kernel,
    out_type=jax.ShapeDtypeStruct((_N_ROWS * _DIM,), jnp.float32),
    mesh=plsc.VectorSubcoreMesh(core_axis_name="c", subcore_axis_name="s"),
    compiler_params=pltpu.CompilerParams(needs_layout_passes=False),
    scratch_types=[
        pltpu.VMEM((_BATCH,), jnp.int32),
        pltpu.VMEM((_WCAP,), jnp.int32),
        pltpu.VMEM((_HCAP,), jnp.int32),
        pltpu.VMEM((_HCAP,), jnp.int32),
        pltpu.VMEM((_DIM,), jnp.float32),
        pltpu.VMEM((_CE,), jnp.float32),
        pltpu.VMEM((_CE,), jnp.float32),
        pltpu.VMEM((_CE,), jnp.float32),
        pltpu.VMEM_SHARED((_BATCH * _DIM,), jnp.float32),
        pltpu.SemaphoreType.DMA,
        pltpu.SemaphoreType.DMA,
        pltpu.SemaphoreType.DMA,
        pltpu.SemaphoreType.DMA,
        pltpu.SemaphoreType.DMA,
        pltpu.SemaphoreType.DMA,
        pltpu.SemaphoreType.DMA,
    ],
)(_mb_body)


def kernel(feats, indexes, bank):
    out = _mb_update(feats.reshape(-1), indexes.astype(jnp.int32),
                     bank.reshape(-1))
    return out.reshape(_N_ROWS, _DIM)


# speedup vs baseline: 1.0215x; 1.0215x over previous
import functools

import jax
import jax.numpy as jnp
from jax import lax
from jax.experimental import pallas as pl
from jax.experimental.pallas import tpu as pltpu
from jax.experimental.pallas import tpu_sc as plsc

_N_ROWS = 1000000
_DIM = 16
_BATCH = 16384
_MOM = 0.5
_NC, _NS, _L = 2, 16, 16
_NW = _NC * _NS
_RPW = _N_ROWS // _NW
_H0 = 16384
_H1MAX = 31256 - _H0
_NGB = _BATCH // _L
_CH = 256
_HCAP = _BATCH + _L
_CBR = 1488
_NCB = 31248 // _CBR


def _bsplat(s):
    return lax.broadcast_in_dim(s, (_L,), ())


def _row_normalize(v):
    s = jnp.sum(v * v)
    sb = _bsplat(s)
    i = plsc.bitcast(sb, jnp.int32)
    i = 0x5F3759DF - lax.shift_right_arithmetic(i, 1)
    y = plsc.bitcast(i, jnp.float32)
    for _ in range(3):
        y = y * (1.5 - 0.5 * sb * y * y)
    norm = sb * y
    return v / (norm + 1e-10)


def _mb_body(feats_hbm, idx_hbm, bank_hbm, out_hbm,
             idxv, wref, hitb, hitrow, tgt,
             fch, och, nch, cb0, cb1,
             sem_ci0, sem_ci1, sem_co0, sem_co1,
             sem_g1, sem_g2, sem_s):
    cid = lax.axis_index("c")
    sid = lax.axis_index("s")
    wid = cid * _NS + sid
    lo = pl.multiple_of((wid * _RPW) // 8 * 8, 8)
    hi = pl.multiple_of(((wid + 1) * _RPW) // 8 * 8, 8)

    iota = lax.iota(jnp.int32, _L)
    neg1 = jnp.full((_L,), -1, jnp.int32)

    pltpu.sync_copy(idx_hbm, idxv)

    def slice_copy():
        bufs = (cb0, cb1)
        isems = (sem_ci0, sem_ci1)
        osems = (sem_co0, sem_co1)
        outs = [None, None]
        ins = [None, None]

        def off(i):
            return pl.multiple_of(lo + i * _CBR, 8)

        ins[0] = pltpu.async_copy(bank_hbm.at[pl.ds(off(0), _CBR)],
                                  cb0, sem_ci0)
        for i in range(_NCB):
            b = i % 2
            if i + 1 < _NCB:
                nb = (i + 1) % 2
                if outs[nb] is not None:
                    outs[nb].wait()
                ins[nb] = pltpu.async_copy(
                    bank_hbm.at[pl.ds(off(i + 1), _CBR)], bufs[nb], isems[nb])
            ins[b].wait()
            outs[b] = pltpu.async_copy(
                bufs[b], out_hbm.at[pl.ds(off(i), _CBR)], osems[b])
        outs[(_NCB - 1) % 2].wait()
        if _NCB >= 2:
            outs[(_NCB - 2) % 2].wait()

        @pl.when(hi - lo > 31248)
        def _tail_copy():
            tl = pl.multiple_of(lo + 31248, 8)
            pltpu.sync_copy(bank_hbm.at[pl.ds(tl, 8)],
                            out_hbm.at[pl.ds(tl, 8)])

    def init_w(_):
        def ib(g, c):
            wref[pl.ds(g * _L, _L)] = neg1
            return c
        lax.fori_loop(0, _H0 // _L, ib, 0)

    def claim_half(hlo, hhi):
        def gb(g, c):
            t = idxv[pl.ds(g * _L, _L)]
            m = (t >= hlo) & (t < hhi)
            local = jnp.where(m, t - hlo, 0)
            bvec = g * _L + iota
            cnt = jnp.sum(jnp.where(m, 1, 0))

            def fast(c):
                plsc.store_scatter(wref, [local], bvec, mask=m)
                return c

            def slow(c):
                for lane in range(_L):
                    ml = m & (iota == lane)
                    plsc.store_scatter(wref, [local], bvec, mask=ml)
                return c

            return lax.cond(
                cnt > 1, slow,
                lambda c: lax.cond(cnt > 0, fast, lambda c: c, c), c)
        lax.fori_loop(0, _NGB, gb, 0)

    def collect_half(hlo, ngroups, cursor):
        def gb(g, cur):
            wv = wref[pl.ds(g * _L, _L)]
            m = wv >= 0
            cnt = jnp.sum(jnp.where(m, 1, 0))
            rowv = hlo + g * _L + iota

            def do(cur):
                plsc.store_compressed(hitb.at[pl.ds(cur, _L)], wv, mask=m)
                plsc.store_compressed(hitrow.at[pl.ds(cur, _L)], rowv, mask=m)
                return cur + cnt

            return lax.cond(cnt > 0, do, lambda c: c, cur)
        return lax.fori_loop(0, ngroups, gb, cursor)

    with jax.named_scope("claim0"):
        init_w(0)
        claim_half(lo, lo + _H0)
        n0 = collect_half(lo, _H0 // _L, 0)
    with jax.named_scope("claim1"):
        init_w(0)
        claim_half(lo + _H0, hi)
        n = collect_half(lo + _H0, (_H1MAX + _L - 1) // _L, n0)

    nchunks = (n + _CH - 1) // _CH

    def fill(_):
        h0v = hitb[pl.ds(0, _L)]
        r0v = hitrow[pl.ds(0, _L)]
        h0 = _bsplat(jnp.sum(jnp.where(iota == 0, h0v, 0)))
        r0 = _bsplat(jnp.sum(jnp.where(iota == 0, r0v, 0)))

        def fb(g, c):
            sl = g * _L + iota
            sel = sl >= n
            hv = hitb[pl.ds(g * _L, _L)]
            rv = hitrow[pl.ds(g * _L, _L)]
            hitb[pl.ds(g * _L, _L)] = jnp.where(sel, h0, hv)
            hitrow[pl.ds(g * _L, _L)] = jnp.where(sel, r0, rv)
            return c
        lax.fori_loop(n // _L, nchunks * (_CH // _L), fb, 0)
        return 0

    with jax.named_scope("fill"):
        lax.cond(n > 0, fill, lambda c: c, 0)

    with jax.named_scope("slicecopy"):
        slice_copy()

    def chunk(c, carry):
        def tc(i, cc):
            tgt[pl.ds(i * _L, _L)] = hitrow[pl.ds(c * _CH + i * _L, _L)]
            return cc
        lax.fori_loop(0, _CH // _L, tc, 0)
        g1 = pltpu.async_copy(feats_hbm.at[hitb.at[pl.ds(c * _CH, _CH)]],
                              fch, sem_g1)
        g2 = pltpu.async_copy(bank_hbm.at[tgt], och, sem_g2)
        g1.wait()
        g2.wait()

        def row(j, cc):
            f = _row_normalize(fch[j, pl.ds(0, _DIM)])
            blended = (1.0 - _MOM) * och[j, pl.ds(0, _DIM)] + _MOM * f
            nch[j, pl.ds(0, _DIM)] = _row_normalize(blended)
            return cc
        lax.fori_loop(0, _CH, row, 0)

        pltpu.async_copy(nch, out_hbm.at[tgt], sem_s).wait()
        return carry
    with jax.named_scope("chunks"):
        lax.fori_loop(0, nchunks, chunk, 0)


_mb_update = functools.partial(
    pl.kernel,
    out_type=jax.ShapeDtypeStruct((_N_ROWS, _DIM), jnp.float32),
    mesh=plsc.VectorSubcoreMesh(core_axis_name="c", subcore_axis_name="s"),
    compiler_params=pltpu.CompilerParams(
        needs_layout_passes=False, use_tc_tiling_on_sc=False),
    scratch_types=[
        pltpu.VMEM((_BATCH,), jnp.int32),
        pltpu.VMEM((_H0,), jnp.int32),
        pltpu.VMEM((_HCAP,), jnp.int32),
        pltpu.VMEM((_HCAP,), jnp.int32),
        pltpu.VMEM((_CH,), jnp.int32),
        pltpu.VMEM((_CH, _DIM), jnp.float32),
        pltpu.VMEM((_CH, _DIM), jnp.float32),
        pltpu.VMEM((_CH, _DIM), jnp.float32),
        pltpu.VMEM((_CBR, _DIM), jnp.float32),
        pltpu.VMEM((_CBR, _DIM), jnp.float32),
        pltpu.SemaphoreType.DMA,
        pltpu.SemaphoreType.DMA,
        pltpu.SemaphoreType.DMA,
        pltpu.SemaphoreType.DMA,
        pltpu.SemaphoreType.DMA,
        pltpu.SemaphoreType.DMA,
        pltpu.SemaphoreType.DMA,
    ],
)(_mb_body)


def kernel(feats, indexes, bank):
    return _mb_update(feats, indexes.astype(jnp.int32), bank)
